# baseline (speedup 1.0000x reference)
import functools

import jax
import jax.numpy as jnp
from jax import lax
from jax.experimental import pallas as pl
from jax.experimental.pallas import tpu as pltpu
from jax.experimental.pallas import tpu_sc as plsc

N_NODE = 10000
H = 256
OUT = 64
E = 160000
NC, NS, L = 2, 16, 16
HALF = N_NODE // NC
TRASH = HALF
ACC_ROWS = HALF + 8
PIECE = 8
PPR = H // PIECE
ACC_P = ACC_ROWS * PPR
ZERO_BLOCKS = ACC_P // 256
CHUNK = 128
FLAT = CHUNK * PPR
SUBT = FLAT // 128
E_PAD = 163840
CHUNKS_PER_TILE = E_PAD // NS // CHUNK
OUT_BLOCKS = HALF * PPR // 256
CNT_N = 10240
CNT_SLICE = CNT_N // NS
REP_ROWS = 10016
ROWS_BLK = 1000


def _cnt_body(dst_hbm, rec_out, cnt_sh, dst_all, cnt_v, red_v, brow):
    c = lax.axis_index("c")
    s = lax.axis_index("s")
    zero16 = jnp.zeros((L,), jnp.float32)
    one16 = jnp.ones((L,), jnp.float32)
    iota = lax.iota(jnp.int32, L)

    pltpu.sync_copy(dst_hbm.at[s], dst_all)

    def zcnt(i, _):
        cnt_v[pl.ds(i * L, L)] = zero16
        return 0
    lax.fori_loop(0, CNT_N // L, zcnt, 0)

    def chunk_body(j, _):
        for i in range(CHUNK // L):
            d = dst_all[j, pl.ds(i * L, L)]
            plsc.addupdate_scatter(cnt_v, [d], one16)
        return 0
    lax.fori_loop(0, CHUNKS_PER_TILE, chunk_body, 0)

    pltpu.sync_copy(cnt_v, cnt_sh.at[s])
    plsc.subcore_barrier()
    r0 = s * CNT_SLICE
    for i in range(CNT_SLICE // L):
        red_v[pl.ds(i * L, L)] = zero16

    def red_body(t, _):
        pltpu.sync_copy(cnt_sh.at[t, pl.ds(r0, CNT_SLICE)], cnt_v.at[pl.ds(0, CNT_SLICE)])
        for i in range(CNT_SLICE // L):
            red_v[pl.ds(i * L, L)] = (red_v[pl.ds(i * L, L)]
                                      + cnt_v[pl.ds(i * L, L)])
        return 0
    lax.fori_loop(0, NS, red_body, 0)

    def rep_body(g, _):
        row0 = r0 + c * 320 + g * 8
        @pl.when(row0 < REP_ROWS)
        def _():
            for r in range(8):
                p = row0 + r - r0
                grp = (p // L) * L
                sl = 1.0 / jnp.maximum(red_v[pl.ds(grp, L)], 1.0)
                sel = jnp.where(iota == (p - grp), 1.0, 0.0)
                rec = jnp.sum(sl * sel)
                rv = jnp.full((L,), rec, jnp.float32)
                for q in range(128 // L):
                    brow[r, pl.ds(q * L, L)] = rv
            pltpu.sync_copy(brow, rec_out.at[pl.ds(row0, 8)])
        return 0
    lax.fori_loop(0, 320 // 8, rep_body, 0)


_sc_counts = functools.partial(
    pl.kernel,
    out_type=jax.ShapeDtypeStruct((REP_ROWS, 128), jnp.float32),
    mesh=plsc.VectorSubcoreMesh(core_axis_name="c", subcore_axis_name="s"),
    compiler_params=pltpu.CompilerParams(needs_layout_passes=False,
                                         use_tc_tiling_on_sc=False),
    scratch_types=[
        pltpu.VMEM_SHARED((NS, CNT_N), jnp.float32),
        pltpu.VMEM((CHUNKS_PER_TILE, CHUNK), jnp.int32),
        pltpu.VMEM((CNT_N,), jnp.float32),
        pltpu.VMEM((CNT_SLICE,), jnp.float32),
        pltpu.VMEM((8, 128), jnp.float32),
    ],
)(_cnt_body)


def _agg_body(h_hbm, src_hbm, dst_hbm, zeros_hbm, s_out,
              acc, src_v, dst_v, spidx, dpidx, rows, zb, gsem, ssem):
    c = lax.axis_index("c")
    s = lax.axis_index("s")
    base = c * HALF
    iota = lax.iota(jnp.int32, L)

    pltpu.sync_copy(zeros_hbm, zb)

    def zero_body(j, _):
        blk = j * NS + s
        @pl.when(blk < ZERO_BLOCKS)
        def _():
            pltpu.sync_copy(zb, acc.at[pl.ds(blk * 256, 256)])
        return 0
    lax.fori_loop(0, (ZERO_BLOCKS + NS - 1) // NS, zero_body, 0)
    plsc.subcore_barrier()

    row16 = [4 * i + iota // 4 for i in range(CHUNK // L)]
    colbase = (iota % 4) * PPR

    def chunk_body(j, _):
        pltpu.sync_copy(src_hbm.at[s, j], src_v)
        pltpu.sync_copy(dst_hbm.at[s, j], dst_v)
        for i in range(CHUNK // L):
            sv = src_v[pl.ds(i * L, L)] * PPR
            d = dst_v[pl.ds(i * L, L)]
            dl = d - base
            ok = (dl >= 0) & (dl < HALF)
            dl = jnp.where(ok, dl, TRASH) * PPR
            for k in range(PPR):
                col = colbase + k
                plsc.store_scatter(spidx, [row16[i], col], sv + k)
                plsc.store_scatter(dpidx, [row16[i], col], dl + k)
        descs = []
        for t in range(SUBT):
            descs.append(pltpu.async_copy(
                h_hbm.at[spidx.at[t]], rows.at[pl.ds(t * 128, 128)], gsem))
        for dsc in descs:
            dsc.wait()
        descs = []
        for t in range(SUBT):
            descs.append(pltpu.async_copy(
                rows.at[pl.ds(t * 128, 128)], acc.at[dpidx.at[t]], ssem,
                add=True))
        for dsc in descs:
            dsc.wait()
        return 0
    lax.fori_loop(0, CHUNKS_PER_TILE, chunk_body, 0)
    plsc.subcore_barrier()

    def out_body(j, _):
        blk = j * NS + s
        @pl.when(blk < OUT_BLOCKS)
        def _():
            pltpu.sync_copy(acc.at[pl.ds(blk * 256, 256)],
                            s_out.at[pl.ds(c * HALF * PPR + blk * 256, 256)])
        return 0
    lax.fori_loop(0, (OUT_BLOCKS + NS - 1) // NS, out_body, 0)


_sc_aggregate = functools.partial(
    pl.kernel,
    out_type=jax.ShapeDtypeStruct((N_NODE * PPR, PIECE), jnp.float32),
    mesh=plsc.VectorSubcoreMesh(core_axis_name="c", subcore_axis_name="s"),
    compiler_params=pltpu.CompilerParams(needs_layout_passes=False,
                                         use_tc_tiling_on_sc=False),
    scratch_types=[
        pltpu.VMEM_SHARED((ACC_P, PIECE), jnp.float32),
        pltpu.VMEM((CHUNK,), jnp.int32),
        pltpu.VMEM((CHUNK,), jnp.int32),
        pltpu.VMEM((SUBT, 128), jnp.int32),
        pltpu.VMEM((SUBT, 128), jnp.int32),
        pltpu.VMEM((FLAT, PIECE), jnp.float32),
        pltpu.VMEM((256, PIECE), jnp.float32),
        pltpu.SemaphoreType.DMA,
        pltpu.SemaphoreType.DMA,
    ],
)(_agg_body)


def _proj_body(xu, wu, bu, xi, wi, bi, hu, hi):
    hu[:] = jnp.dot(xu[:], wu[:], preferred_element_type=jnp.float32) + bu[:]
    hi[:] = jnp.dot(xi[:], wi[:], preferred_element_type=jnp.float32) + bi[:]


def _layer0_body(sit, rit, hi0, wl_i, bl_i, wr_i,
                 sus, rus, hu0, wl_u, bl_u, wr_u, hi1, hu1):
    mi = sit[:] * rit[:, 0:1]
    hi1[:] = jax.nn.relu(
        jnp.dot(mi, wl_i[:], preferred_element_type=jnp.float32) + bl_i[:]
        + jnp.dot(hi0[:], wr_i[:], preferred_element_type=jnp.float32))
    mu = sus[:] * rus[:, 0:1]
    hu1[:] = jax.nn.relu(
        jnp.dot(mu, wl_u[:], preferred_element_type=jnp.float32) + bl_u[:]
        + jnp.dot(hu0[:], wr_u[:], preferred_element_type=jnp.float32))


def _final_body(sus2, rus, hu1, wl, bl, wr, wh1, bh1, wh2, bh2, out):
    mu = sus2[:] * rus[:, 0:1]
    hu2 = (jnp.dot(mu, wl[:], preferred_element_type=jnp.float32) + bl[:]
           + jnp.dot(hu1[:], wr[:], preferred_element_type=jnp.float32))
    t = jax.nn.relu(
        jnp.dot(hu2, wh1[:], preferred_element_type=jnp.float32) + bh1[:])
    out[:] = jnp.dot(t, wh2[:], preferred_element_type=jnp.float32) + bh2[:]


def _row_spec(cols):
    return pl.BlockSpec((ROWS_BLK, cols), lambda i: (i, 0))


def _full_spec(r, cols):
    return pl.BlockSpec((r, cols), lambda i: (0, 0))


_GRID = N_NODE // ROWS_BLK


def _tc_proj(xu, wu, bu, xi, wi, bi):
    return pl.pallas_call(
        _proj_body,
        grid=(_GRID,),
        in_specs=[_row_spec(256), _full_spec(256, H), _full_spec(1, H),
                  _row_spec(128), _full_spec(128, H), _full_spec(1, H)],
        out_specs=(_row_spec(H), _row_spec(H)),
        out_shape=(jax.ShapeDtypeStruct((N_NODE, H), jnp.float32),
                   jax.ShapeDtypeStruct((N_NODE, H), jnp.float32)),
    )(xu, wu, bu, xi, wi, bi)


def _tc_layer0(sit, rit, hi0, wl_i, bl_i, wr_i,
               sus, rus, hu0, wl_u, bl_u, wr_u):
    w = _full_spec(H, H)
    b = _full_spec(1, H)
    return pl.pallas_call(
        _layer0_body,
        grid=(_GRID,),
        in_specs=[_row_spec(H), _row_spec(128), _row_spec(H), w, b, w,
                  _row_spec(H), _row_spec(128), _row_spec(H), w, b, w],
        out_specs=(_row_spec(H), _row_spec(H)),
        out_shape=(jax.ShapeDtypeStruct((N_NODE, H), jnp.float32),
                   jax.ShapeDtypeStruct((N_NODE, H), jnp.float32)),
    )(sit, rit, hi0, wl_i, bl_i, wr_i, sus, rus, hu0, wl_u, bl_u, wr_u)


def _tc_final(sus2, rus, hu1, wl, bl, wr, wh1, bh1, wh2, bh2):
    w = _full_spec(H, H)
    b = _full_spec(1, H)
    return pl.pallas_call(
        _final_body,
        grid=(_GRID,),
        in_specs=[_row_spec(H), _row_spec(128), _row_spec(H), w, b, w,
                  w, b, _full_spec(H, OUT), _full_spec(1, OUT)],
        out_specs=_row_spec(OUT),
        out_shape=jax.ShapeDtypeStruct((N_NODE, OUT), jnp.float32),
    )(sus2, rus, hu1, wl, bl, wr, wh1, bh1, wh2, bh2)


def kernel(x_user, x_item, W_proj_user, b_proj_user, W_proj_item, b_proj_item,
           W_l0_ui, b_l0_ui, W_r0_ui, W_l0_iu, b_l0_iu, W_r0_iu,
           W_l1_ui, b_l1_ui, W_r1_ui, W_l1_iu, b_l1_iu, W_r1_iu,
           W_head1, b_head1, W_head2, b_head2, edge_index_ui, edge_index_iu):
    pad_n = E_PAD - E
    shape3 = (NS, CHUNKS_PER_TILE, CHUNK)
    src_ui = jnp.concatenate(
        [edge_index_ui[0], jnp.zeros((pad_n,), jnp.int32)]).reshape(shape3)
    dst_ui = jnp.concatenate(
        [edge_index_ui[1],
         jnp.full((pad_n,), N_NODE, jnp.int32)]).reshape(shape3)
    src_iu = jnp.concatenate(
        [edge_index_iu[0], jnp.zeros((pad_n,), jnp.int32)]).reshape(shape3)
    dst_iu = jnp.concatenate(
        [edge_index_iu[1],
         jnp.full((pad_n,), N_NODE, jnp.int32)]).reshape(shape3)
    zeros_blk = jnp.zeros((256, PIECE), jnp.float32)

    rec_it = _sc_counts(dst_ui)
    rec_us = _sc_counts(dst_iu)

    hu0, hi0 = _tc_proj(x_user, W_proj_user, b_proj_user.reshape(1, H),
                        x_item, W_proj_item, b_proj_item.reshape(1, H))

    def agg(h, src3, dst3):
        s_p = _sc_aggregate(h.reshape(N_NODE * PPR, PIECE), src3, dst3,
                            zeros_blk)
        return s_p.reshape(N_NODE, H)

    s_it = agg(hu0, src_ui, dst_ui)
    s_us = agg(hi0, src_iu, dst_iu)

    hi1, hu1 = _tc_layer0(s_it, rec_it, hi0, W_l0_ui, b_l0_ui.reshape(1, H),
                          W_r0_ui, s_us, rec_us, hu0, W_l0_iu,
                          b_l0_iu.reshape(1, H), W_r0_iu)

    s_us2 = agg(hi1, src_iu, dst_iu)

    return _tc_final(s_us2, rec_us, hu1, W_l1_iu, b_l1_iu.reshape(1, H),
                     W_r1_iu, W_head1, b_head1.reshape(1, H),
                     W_head2, b_head2.reshape(1, OUT))


# speedup vs baseline: 1.0167x; 1.0167x over previous
import functools

import jax
import jax.numpy as jnp
from jax import lax
from jax.experimental import pallas as pl
from jax.experimental.pallas import tpu as pltpu
from jax.experimental.pallas import tpu_sc as plsc

N_NODE = 10000
H = 256
OUT = 64
E = 160000
NC, NS, L = 2, 16, 16
HALF = N_NODE // NC
TRASH = HALF
ACC_ROWS = HALF + 8
PIECE = 8
PPR = H // PIECE
ACC_P = ACC_ROWS * PPR
ZERO_BLOCKS = ACC_P // 256
CHUNK = 64
FLAT = CHUNK * PPR
SUBT = FLAT // 128
E_PAD = 163840
CHUNKS_PER_TILE = E_PAD // NS // CHUNK
OUT_BLOCKS = HALF * PPR // 256
CNT_N = 10240
CNT_SLICE = CNT_N // NS
REP_ROWS = 10016
ROWS_BLK = 1000


def _cnt_body(dst_hbm, rec_out, cnt_sh, dst_all, cnt_v, red_v, brow):
    c = lax.axis_index("c")
    s = lax.axis_index("s")
    zero16 = jnp.zeros((L,), jnp.float32)
    one16 = jnp.ones((L,), jnp.float32)
    iota = lax.iota(jnp.int32, L)

    pltpu.sync_copy(dst_hbm.at[s], dst_all)

    def zcnt(i, _):
        cnt_v[pl.ds(i * L, L)] = zero16
        return 0
    lax.fori_loop(0, CNT_N // L, zcnt, 0)

    def chunk_body(j, _):
        for i in range(CHUNK // L):
            d = dst_all[j, pl.ds(i * L, L)]
            plsc.addupdate_scatter(cnt_v, [d], one16)
        return 0
    lax.fori_loop(0, CHUNKS_PER_TILE, chunk_body, 0)

    pltpu.sync_copy(cnt_v, cnt_sh.at[s])
    plsc.subcore_barrier()
    r0 = s * CNT_SLICE
    for i in range(CNT_SLICE // L):
        red_v[pl.ds(i * L, L)] = zero16

    def red_body(t, _):
        pltpu.sync_copy(cnt_sh.at[t, pl.ds(r0, CNT_SLICE)], cnt_v.at[pl.ds(0, CNT_SLICE)])
        for i in range(CNT_SLICE // L):
            red_v[pl.ds(i * L, L)] = (red_v[pl.ds(i * L, L)]
                                      + cnt_v[pl.ds(i * L, L)])
        return 0
    lax.fori_loop(0, NS, red_body, 0)

    def rep_body(g, _):
        row0 = r0 + c * 320 + g * 8
        @pl.when(row0 < REP_ROWS)
        def _():
            for r in range(8):
                p = row0 + r - r0
                grp = (p // L) * L
                sl = 1.0 / jnp.maximum(red_v[pl.ds(grp, L)], 1.0)
                sel = jnp.where(iota == (p - grp), 1.0, 0.0)
                rec = jnp.sum(sl * sel)
                rv = jnp.full((L,), rec, jnp.float32)
                for q in range(128 // L):
                    brow[r, pl.ds(q * L, L)] = rv
            pltpu.sync_copy(brow, rec_out.at[pl.ds(row0, 8)])
        return 0
    lax.fori_loop(0, 320 // 8, rep_body, 0)


_sc_counts = functools.partial(
    pl.kernel,
    out_type=jax.ShapeDtypeStruct((REP_ROWS, 128), jnp.float32),
    mesh=plsc.VectorSubcoreMesh(core_axis_name="c", subcore_axis_name="s"),
    compiler_params=pltpu.CompilerParams(needs_layout_passes=False,
                                         use_tc_tiling_on_sc=False),
    scratch_types=[
        pltpu.VMEM_SHARED((NS, CNT_N), jnp.float32),
        pltpu.VMEM((CHUNKS_PER_TILE, CHUNK), jnp.int32),
        pltpu.VMEM((CNT_N,), jnp.float32),
        pltpu.VMEM((CNT_SLICE,), jnp.float32),
        pltpu.VMEM((8, 128), jnp.float32),
    ],
)(_cnt_body)


def _agg_body(h_hbm, src_hbm, dst_hbm, zeros_hbm, s_out,
              acc, src_v, dst_v, spidx, dpidx, rows, zb, gsem, ssem):
    c = lax.axis_index("c")
    s = lax.axis_index("s")
    base = c * HALF
    iota = lax.iota(jnp.int32, L)

    pltpu.sync_copy(zeros_hbm, zb)

    def zero_body(j, _):
        blk = j * NS + s
        @pl.when(blk < ZERO_BLOCKS)
        def _():
            pltpu.sync_copy(zb, acc.at[pl.ds(blk * 256, 256)])
        return 0
    lax.fori_loop(0, (ZERO_BLOCKS + NS - 1) // NS, zero_body, 0)
    plsc.subcore_barrier()

    row16 = [4 * i + iota // 4 for i in range(CHUNK // L)]
    colbase = (iota % 4) * PPR

    def build_and_gather(j, slot):
        pltpu.sync_copy(src_hbm.at[s, j], src_v)
        pltpu.sync_copy(dst_hbm.at[s, j], dst_v)
        for i in range(CHUNK // L):
            sv = src_v[pl.ds(i * L, L)] * PPR
            d = dst_v[pl.ds(i * L, L)]
            dl = d - base
            ok = (dl >= 0) & (dl < HALF)
            dl = jnp.where(ok, dl, TRASH) * PPR
            for k in range(PPR):
                col = colbase + k
                plsc.store_scatter(spidx.at[slot], [row16[i], col], sv + k)
                plsc.store_scatter(dpidx.at[slot], [row16[i], col], dl + k)
        return [pltpu.async_copy(h_hbm.at[spidx.at[slot, t]],
                                 rows.at[slot, pl.ds(t * 128, 128)], gsem)
                for t in range(SUBT)]

    def drain(descs):
        for dsc in descs:
            dsc.wait()

    def fire_scatter(slot):
        return [pltpu.async_copy(rows.at[slot, pl.ds(t * 128, 128)],
                                 acc.at[dpidx.at[slot, t]], ssem, add=True)
                for t in range(SUBT)]

    drain(build_and_gather(0, 0))

    def pair_body(jj, _):
        for b in range(2):
            j = jj * 2 + b
            sdescs = fire_scatter(b)
            @pl.when(j + 1 < CHUNKS_PER_TILE)
            def _():
                drain(build_and_gather(j + 1, 1 - b))
            drain(sdescs)
        return 0
    lax.fori_loop(0, CHUNKS_PER_TILE // 2, pair_body, 0)
    plsc.subcore_barrier()

    def out_body(j, _):
        blk = j * NS + s
        @pl.when(blk < OUT_BLOCKS)
        def _():
            pltpu.sync_copy(acc.at[pl.ds(blk * 256, 256)],
                            s_out.at[pl.ds(c * HALF * PPR + blk * 256, 256)])
        return 0
    lax.fori_loop(0, (OUT_BLOCKS + NS - 1) // NS, out_body, 0)


_sc_aggregate = functools.partial(
    pl.kernel,
    out_type=jax.ShapeDtypeStruct((N_NODE * PPR, PIECE), jnp.float32),
    mesh=plsc.VectorSubcoreMesh(core_axis_name="c", subcore_axis_name="s"),
    compiler_params=pltpu.CompilerParams(needs_layout_passes=False,
                                         use_tc_tiling_on_sc=False),
    scratch_types=[
        pltpu.VMEM_SHARED((ACC_P, PIECE), jnp.float32),
        pltpu.VMEM((CHUNK,), jnp.int32),
        pltpu.VMEM((CHUNK,), jnp.int32),
        pltpu.VMEM((2, SUBT, 128), jnp.int32),
        pltpu.VMEM((2, SUBT, 128), jnp.int32),
        pltpu.VMEM((2, FLAT, PIECE), jnp.float32),
        pltpu.VMEM((256, PIECE), jnp.float32),
        pltpu.SemaphoreType.DMA,
        pltpu.SemaphoreType.DMA,
    ],
)(_agg_body)


def _proj_body(xu, wu, bu, xi, wi, bi, hu, hi):
    hu[:] = jnp.dot(xu[:], wu[:], preferred_element_type=jnp.float32) + bu[:]
    hi[:] = jnp.dot(xi[:], wi[:], preferred_element_type=jnp.float32) + bi[:]


def _layer0_body(sit, rit, hi0, wl_i, bl_i, wr_i,
                 sus, rus, hu0, wl_u, bl_u, wr_u, hi1, hu1):
    mi = sit[:] * rit[:, 0:1]
    hi1[:] = jax.nn.relu(
        jnp.dot(mi, wl_i[:], preferred_element_type=jnp.float32) + bl_i[:]
        + jnp.dot(hi0[:], wr_i[:], preferred_element_type=jnp.float32))
    mu = sus[:] * rus[:, 0:1]
    hu1[:] = jax.nn.relu(
        jnp.dot(mu, wl_u[:], preferred_element_type=jnp.float32) + bl_u[:]
        + jnp.dot(hu0[:], wr_u[:], preferred_element_type=jnp.float32))


def _final_body(sus2, rus, hu1, wl, bl, wr, wh1, bh1, wh2, bh2, out):
    mu = sus2[:] * rus[:, 0:1]
    hu2 = (jnp.dot(mu, wl[:], preferred_element_type=jnp.float32) + bl[:]
           + jnp.dot(hu1[:], wr[:], preferred_element_type=jnp.float32))
    t = jax.nn.relu(
        jnp.dot(hu2, wh1[:], preferred_element_type=jnp.float32) + bh1[:])
    out[:] = jnp.dot(t, wh2[:], preferred_element_type=jnp.float32) + bh2[:]


def _row_spec(cols):
    return pl.BlockSpec((ROWS_BLK, cols), lambda i: (i, 0))


def _full_spec(r, cols):
    return pl.BlockSpec((r, cols), lambda i: (0, 0))


_GRID = N_NODE // ROWS_BLK


def _tc_proj(xu, wu, bu, xi, wi, bi):
    return pl.pallas_call(
        _proj_body,
        grid=(_GRID,),
        in_specs=[_row_spec(256), _full_spec(256, H), _full_spec(1, H),
                  _row_spec(128), _full_spec(128, H), _full_spec(1, H)],
        out_specs=(_row_spec(H), _row_spec(H)),
        out_shape=(jax.ShapeDtypeStruct((N_NODE, H), jnp.float32),
                   jax.ShapeDtypeStruct((N_NODE, H), jnp.float32)),
    )(xu, wu, bu, xi, wi, bi)


def _tc_layer0(sit, rit, hi0, wl_i, bl_i, wr_i,
               sus, rus, hu0, wl_u, bl_u, wr_u):
    w = _full_spec(H, H)
    b = _full_spec(1, H)
    return pl.pallas_call(
        _layer0_body,
        grid=(_GRID,),
        in_specs=[_row_spec(H), _row_spec(128), _row_spec(H), w, b, w,
                  _row_spec(H), _row_spec(128), _row_spec(H), w, b, w],
        out_specs=(_row_spec(H), _row_spec(H)),
        out_shape=(jax.ShapeDtypeStruct((N_NODE, H), jnp.float32),
                   jax.ShapeDtypeStruct((N_NODE, H), jnp.float32)),
    )(sit, rit, hi0, wl_i, bl_i, wr_i, sus, rus, hu0, wl_u, bl_u, wr_u)


def _tc_final(sus2, rus, hu1, wl, bl, wr, wh1, bh1, wh2, bh2):
    w = _full_spec(H, H)
    b = _full_spec(1, H)
    return pl.pallas_call(
        _final_body,
        grid=(_GRID,),
        in_specs=[_row_spec(H), _row_spec(128), _row_spec(H), w, b, w,
                  w, b, _full_spec(H, OUT), _full_spec(1, OUT)],
        out_specs=_row_spec(OUT),
        out_shape=jax.ShapeDtypeStruct((N_NODE, OUT), jnp.float32),
    )(sus2, rus, hu1, wl, bl, wr, wh1, bh1, wh2, bh2)


def kernel(x_user, x_item, W_proj_user, b_proj_user, W_proj_item, b_proj_item,
           W_l0_ui, b_l0_ui, W_r0_ui, W_l0_iu, b_l0_iu, W_r0_iu,
           W_l1_ui, b_l1_ui, W_r1_ui, W_l1_iu, b_l1_iu, W_r1_iu,
           W_head1, b_head1, W_head2, b_head2, edge_index_ui, edge_index_iu):
    pad_n = E_PAD - E
    shape3 = (NS, CHUNKS_PER_TILE, CHUNK)
    src_ui = jnp.concatenate(
        [edge_index_ui[0], jnp.zeros((pad_n,), jnp.int32)]).reshape(shape3)
    dst_ui = jnp.concatenate(
        [edge_index_ui[1],
         jnp.full((pad_n,), N_NODE, jnp.int32)]).reshape(shape3)
    src_iu = jnp.concatenate(
        [edge_index_iu[0], jnp.zeros((pad_n,), jnp.int32)]).reshape(shape3)
    dst_iu = jnp.concatenate(
        [edge_index_iu[1],
         jnp.full((pad_n,), N_NODE, jnp.int32)]).reshape(shape3)
    zeros_blk = jnp.zeros((256, PIECE), jnp.float32)

    rec_it = _sc_counts(dst_ui)
    rec_us = _sc_counts(dst_iu)

    hu0, hi0 = _tc_proj(x_user, W_proj_user, b_proj_user.reshape(1, H),
                        x_item, W_proj_item, b_proj_item.reshape(1, H))

    def agg(h, src3, dst3):
        s_p = _sc_aggregate(h.reshape(N_NODE * PPR, PIECE), src3, dst3,
                            zeros_blk)
        return s_p.reshape(N_NODE, H)

    s_it = agg(hu0, src_ui, dst_ui)
    s_us = agg(hi0, src_iu, dst_iu)

    hi1, hu1 = _tc_layer0(s_it, rec_it, hi0, W_l0_ui, b_l0_ui.reshape(1, H),
                          W_r0_ui, s_us, rec_us, hu0, W_l0_iu,
                          b_l0_iu.reshape(1, H), W_r0_iu)

    s_us2 = agg(hi1, src_iu, dst_iu)

    return _tc_final(s_us2, rec_us, hu1, W_l1_iu, b_l1_iu.reshape(1, H),
                     W_r1_iu, W_head1, b_head1.reshape(1, H),
                     W_head2, b_head2.reshape(1, OUT))


# speedup vs baseline: 1.2357x; 1.2153x over previous
import functools

import jax
import jax.numpy as jnp
from jax import lax
from jax.experimental import pallas as pl
from jax.experimental.pallas import tpu as pltpu
from jax.experimental.pallas import tpu_sc as plsc

N_NODE = 10000
H = 256
OUT = 64
E = 160000
NC, NS, L = 2, 16, 16
HALF = N_NODE // NC
TRASH = HALF
ACC_ROWS = HALF + 8
PIECE = 8
PPR = H // PIECE
ACC_P = ACC_ROWS * PPR
ZERO_BLOCKS = ACC_P // 256
CHUNK = 64
FLAT = CHUNK * PPR
SUBT = FLAT // 128
E_PAD = 163840
CHUNKS_PER_TILE = E_PAD // NS // CHUNK
OUT_BLOCKS = HALF * PPR // 256
CNT_N = 10240
CNT_SLICE = CNT_N // NS
REP_ROWS = 10016
ROWS_BLK = 1000


def _cnt_body(dst_hbm, rec_out, cnt_sh, dst_all, cnt_v, red_v, brow):
    c = lax.axis_index("c")
    s = lax.axis_index("s")
    zero16 = jnp.zeros((L,), jnp.float32)
    one16 = jnp.ones((L,), jnp.float32)
    iota = lax.iota(jnp.int32, L)

    pltpu.sync_copy(dst_hbm.at[s], dst_all)

    def zcnt(i, _):
        cnt_v[pl.ds(i * L, L)] = zero16
        return 0
    lax.fori_loop(0, CNT_N // L, zcnt, 0)

    def chunk_body(j, _):
        for i in range(CHUNK // L):
            d = dst_all[j, pl.ds(i * L, L)]
            plsc.addupdate_scatter(cnt_v, [d], one16)
        return 0
    lax.fori_loop(0, CHUNKS_PER_TILE, chunk_body, 0)

    pltpu.sync_copy(cnt_v, cnt_sh.at[s])
    plsc.subcore_barrier()
    r0 = s * CNT_SLICE
    for i in range(CNT_SLICE // L):
        red_v[pl.ds(i * L, L)] = zero16

    def red_body(t, _):
        pltpu.sync_copy(cnt_sh.at[t, pl.ds(r0, CNT_SLICE)], cnt_v.at[pl.ds(0, CNT_SLICE)])
        for i in range(CNT_SLICE // L):
            red_v[pl.ds(i * L, L)] = (red_v[pl.ds(i * L, L)]
                                      + cnt_v[pl.ds(i * L, L)])
        return 0
    lax.fori_loop(0, NS, red_body, 0)

    def rep_body(g, _):
        row0 = r0 + c * 320 + g * 8
        @pl.when(row0 < REP_ROWS)
        def _():
            for r in range(8):
                p = row0 + r - r0
                grp = (p // L) * L
                sl = 1.0 / jnp.maximum(red_v[pl.ds(grp, L)], 1.0)
                sel = jnp.where(iota == (p - grp), 1.0, 0.0)
                rec = jnp.sum(sl * sel)
                rv = jnp.full((L,), rec, jnp.float32)
                for q in range(128 // L):
                    brow[r, pl.ds(q * L, L)] = rv
            pltpu.sync_copy(brow, rec_out.at[pl.ds(row0, 8)])
        return 0
    lax.fori_loop(0, 320 // 8, rep_body, 0)


_sc_counts = functools.partial(
    pl.kernel,
    out_type=jax.ShapeDtypeStruct((REP_ROWS, 128), jnp.float32),
    mesh=plsc.VectorSubcoreMesh(core_axis_name="c", subcore_axis_name="s"),
    compiler_params=pltpu.CompilerParams(needs_layout_passes=False,
                                         use_tc_tiling_on_sc=False),
    scratch_types=[
        pltpu.VMEM_SHARED((NS, CNT_N), jnp.float32),
        pltpu.VMEM((CHUNKS_PER_TILE, CHUNK), jnp.int32),
        pltpu.VMEM((CNT_N,), jnp.float32),
        pltpu.VMEM((CNT_SLICE,), jnp.float32),
        pltpu.VMEM((8, 128), jnp.float32),
    ],
)(_cnt_body)


SLICE_E = E_PAD // (NC * NS)
CAP = SLICE_E + CHUNK


def _bucket_body(src_hbm, dst_hbm, bsrc_out, bdst_out, lens_out,
                 sv_all, dv_all, lsrc, ldst, hsrc, hdst, lenv):
    c = lax.axis_index("c")
    s = lax.axis_index("s")
    w = 2 * s + c
    zero16i = jnp.zeros((L,), jnp.int32)
    pad16 = jnp.full((L,), N_NODE, jnp.int32)

    pltpu.sync_copy(src_hbm.at[pl.ds(w * SLICE_E, SLICE_E)], sv_all)
    pltpu.sync_copy(dst_hbm.at[pl.ds(w * SLICE_E, SLICE_E)], dv_all)

    def grp(i, carry):
        lo, hi = carry
        sv = sv_all[pl.ds(i * L, L)]
        dv = dv_all[pl.ds(i * L, L)]
        mlow = dv < HALF
        plsc.store_compressed(lsrc.at[pl.ds(lo, L)], sv, mask=mlow)
        plsc.store_compressed(ldst.at[pl.ds(lo, L)], dv, mask=mlow)
        n_lo = plsc.all_reduce_population_count(mlow)
        mhigh = jnp.logical_not(mlow)
        plsc.store_compressed(hsrc.at[pl.ds(hi, L)], sv, mask=mhigh)
        plsc.store_compressed(hdst.at[pl.ds(hi, L)], dv, mask=mhigh)
        return (lo + jnp.max(n_lo), hi + (L - jnp.max(n_lo)))
    lo, hi = lax.fori_loop(0, SLICE_E // L, grp, (0, 0))

    def pad_tail(off, srcl, dstl):
        tgt = ((off + CHUNK - 1) // CHUNK) * CHUNK
        for i in range(CHUNK // L):
            @pl.when(off + i * L < tgt)
            def _():
                srcl[pl.ds(off + i * L, L)] = zero16i
                dstl[pl.ds(off + i * L, L)] = pad16
        return tgt // CHUNK
    nch_lo = pad_tail(lo, lsrc, ldst)
    nch_hi = pad_tail(hi, hsrc, hdst)

    pltpu.sync_copy(lsrc, bsrc_out.at[0, w])
    pltpu.sync_copy(ldst, bdst_out.at[0, w])
    pltpu.sync_copy(hsrc, bsrc_out.at[1, w])
    pltpu.sync_copy(hdst, bdst_out.at[1, w])
    lenv[pl.ds(0, L)] = jnp.full((L,), nch_lo, jnp.int32)
    lenv[pl.ds(L, L)] = jnp.full((L,), nch_hi, jnp.int32)
    pltpu.sync_copy(lenv.at[pl.ds(0, L)], lens_out.at[0, w])
    pltpu.sync_copy(lenv.at[pl.ds(L, L)], lens_out.at[1, w])


_sc_bucket = functools.partial(
    pl.kernel,
    out_type=(jax.ShapeDtypeStruct((NC, NC * NS, CAP), jnp.int32),
              jax.ShapeDtypeStruct((NC, NC * NS, CAP), jnp.int32),
              jax.ShapeDtypeStruct((NC, NC * NS, L), jnp.int32)),
    mesh=plsc.VectorSubcoreMesh(core_axis_name="c", subcore_axis_name="s"),
    compiler_params=pltpu.CompilerParams(needs_layout_passes=False,
                                         use_tc_tiling_on_sc=False),
    scratch_types=[
        pltpu.VMEM((SLICE_E,), jnp.int32),
        pltpu.VMEM((SLICE_E,), jnp.int32),
        pltpu.VMEM((CAP,), jnp.int32),
        pltpu.VMEM((CAP,), jnp.int32),
        pltpu.VMEM((CAP,), jnp.int32),
        pltpu.VMEM((CAP,), jnp.int32),
        pltpu.VMEM((2 * L,), jnp.int32),
    ],
)(_bucket_body)


def _agg_body(h_hbm, bsrc, bdst, lens, zeros_hbm, s_out,
              acc, src_v, dst_v, spidx, dpidx, rows, zb, lenv, gsem, ssem):
    c = lax.axis_index("c")
    s = lax.axis_index("s")
    base = c * HALF
    iota = lax.iota(jnp.int32, L)

    pltpu.sync_copy(zeros_hbm, zb)

    def zero_body(j, _):
        blk = j * NS + s
        @pl.when(blk < ZERO_BLOCKS)
        def _():
            pltpu.sync_copy(zb, acc.at[pl.ds(blk * 256, 256)])
        return 0
    lax.fori_loop(0, (ZERO_BLOCKS + NS - 1) // NS, zero_body, 0)
    plsc.subcore_barrier()

    row16 = [4 * i + iota // 4 for i in range(CHUNK // L)]
    colbase = (iota % 4) * PPR

    def sublist(w):
        pltpu.sync_copy(lens.at[c, w], lenv)
        nch = jnp.max(lenv[pl.ds(0, L)])

        def chunk_body(j, _):
            pltpu.sync_copy(bsrc.at[c, w, pl.ds(j * CHUNK, CHUNK)], src_v)
            pltpu.sync_copy(bdst.at[c, w, pl.ds(j * CHUNK, CHUNK)], dst_v)
            for i in range(CHUNK // L):
                sv = src_v[pl.ds(i * L, L)] * PPR
                d = dst_v[pl.ds(i * L, L)]
                dl = d - base
                ok = (dl >= 0) & (dl < HALF)
                dl = jnp.where(ok, dl, TRASH) * PPR
                for k in range(PPR):
                    col = colbase + k
                    plsc.store_scatter(spidx, [row16[i], col], sv + k)
                    plsc.store_scatter(dpidx, [row16[i], col], dl + k)
            gd = [pltpu.async_copy(h_hbm.at[spidx.at[t]],
                                   rows.at[pl.ds(t * 128, 128)], gsem)
                  for t in range(SUBT)]
            for dsc in gd:
                dsc.wait()
            sd = [pltpu.async_copy(rows.at[pl.ds(t * 128, 128)],
                                   acc.at[dpidx.at[t]], ssem, add=True)
                  for t in range(SUBT)]
            for dsc in sd:
                dsc.wait()
            return 0
        lax.fori_loop(0, nch, chunk_body, 0)

    sublist(2 * s)
    sublist(2 * s + 1)
    plsc.subcore_barrier()

    def out_body(j, _):
        blk = j * NS + s
        @pl.when(blk < OUT_BLOCKS)
        def _():
            pltpu.sync_copy(acc.at[pl.ds(blk * 256, 256)],
                            s_out.at[pl.ds(c * HALF * PPR + blk * 256, 256)])
        return 0
    lax.fori_loop(0, (OUT_BLOCKS + NS - 1) // NS, out_body, 0)


_sc_aggregate = functools.partial(
    pl.kernel,
    out_type=jax.ShapeDtypeStruct((N_NODE * PPR, PIECE), jnp.float32),
    mesh=plsc.VectorSubcoreMesh(core_axis_name="c", subcore_axis_name="s"),
    compiler_params=pltpu.CompilerParams(needs_layout_passes=False,
                                         use_tc_tiling_on_sc=False),
    scratch_types=[
        pltpu.VMEM_SHARED((ACC_P, PIECE), jnp.float32),
        pltpu.VMEM((CHUNK,), jnp.int32),
        pltpu.VMEM((CHUNK,), jnp.int32),
        pltpu.VMEM((SUBT, 128), jnp.int32),
        pltpu.VMEM((SUBT, 128), jnp.int32),
        pltpu.VMEM((FLAT, PIECE), jnp.float32),
        pltpu.VMEM((256, PIECE), jnp.float32),
        pltpu.VMEM((L,), jnp.int32),
        pltpu.SemaphoreType.DMA,
        pltpu.SemaphoreType.DMA,
    ],
)(_agg_body)


def _proj_body(xu, wu, bu, xi, wi, bi, hu, hi):
    hu[:] = jnp.dot(xu[:], wu[:], preferred_element_type=jnp.float32) + bu[:]
    hi[:] = jnp.dot(xi[:], wi[:], preferred_element_type=jnp.float32) + bi[:]


def _layer0_body(sit, rit, hi0, wl_i, bl_i, wr_i,
                 sus, rus, hu0, wl_u, bl_u, wr_u, hi1, hu1):
    mi = sit[:] * rit[:, 0:1]
    hi1[:] = jax.nn.relu(
        jnp.dot(mi, wl_i[:], preferred_element_type=jnp.float32) + bl_i[:]
        + jnp.dot(hi0[:], wr_i[:], preferred_element_type=jnp.float32))
    mu = sus[:] * rus[:, 0:1]
    hu1[:] = jax.nn.relu(
        jnp.dot(mu, wl_u[:], preferred_element_type=jnp.float32) + bl_u[:]
        + jnp.dot(hu0[:], wr_u[:], preferred_element_type=jnp.float32))


def _final_body(sus2, rus, hu1, wl, bl, wr, wh1, bh1, wh2, bh2, out):
    mu = sus2[:] * rus[:, 0:1]
    hu2 = (jnp.dot(mu, wl[:], preferred_element_type=jnp.float32) + bl[:]
           + jnp.dot(hu1[:], wr[:], preferred_element_type=jnp.float32))
    t = jax.nn.relu(
        jnp.dot(hu2, wh1[:], preferred_element_type=jnp.float32) + bh1[:])
    out[:] = jnp.dot(t, wh2[:], preferred_element_type=jnp.float32) + bh2[:]


def _row_spec(cols):
    return pl.BlockSpec((ROWS_BLK, cols), lambda i: (i, 0))


def _full_spec(r, cols):
    return pl.BlockSpec((r, cols), lambda i: (0, 0))


_GRID = N_NODE // ROWS_BLK


def _tc_proj(xu, wu, bu, xi, wi, bi):
    return pl.pallas_call(
        _proj_body,
        grid=(_GRID,),
        in_specs=[_row_spec(256), _full_spec(256, H), _full_spec(1, H),
                  _row_spec(128), _full_spec(128, H), _full_spec(1, H)],
        out_specs=(_row_spec(H), _row_spec(H)),
        out_shape=(jax.ShapeDtypeStruct((N_NODE, H), jnp.float32),
                   jax.ShapeDtypeStruct((N_NODE, H), jnp.float32)),
    )(xu, wu, bu, xi, wi, bi)


def _tc_layer0(sit, rit, hi0, wl_i, bl_i, wr_i,
               sus, rus, hu0, wl_u, bl_u, wr_u):
    w = _full_spec(H, H)
    b = _full_spec(1, H)
    return pl.pallas_call(
        _layer0_body,
        grid=(_GRID,),
        in_specs=[_row_spec(H), _row_spec(128), _row_spec(H), w, b, w,
                  _row_spec(H), _row_spec(128), _row_spec(H), w, b, w],
        out_specs=(_row_spec(H), _row_spec(H)),
        out_shape=(jax.ShapeDtypeStruct((N_NODE, H), jnp.float32),
                   jax.ShapeDtypeStruct((N_NODE, H), jnp.float32)),
    )(sit, rit, hi0, wl_i, bl_i, wr_i, sus, rus, hu0, wl_u, bl_u, wr_u)


def _tc_final(sus2, rus, hu1, wl, bl, wr, wh1, bh1, wh2, bh2):
    w = _full_spec(H, H)
    b = _full_spec(1, H)
    return pl.pallas_call(
        _final_body,
        grid=(_GRID,),
        in_specs=[_row_spec(H), _row_spec(128), _row_spec(H), w, b, w,
                  w, b, _full_spec(H, OUT), _full_spec(1, OUT)],
        out_specs=_row_spec(OUT),
        out_shape=jax.ShapeDtypeStruct((N_NODE, OUT), jnp.float32),
    )(sus2, rus, hu1, wl, bl, wr, wh1, bh1, wh2, bh2)


def kernel(x_user, x_item, W_proj_user, b_proj_user, W_proj_item, b_proj_item,
           W_l0_ui, b_l0_ui, W_r0_ui, W_l0_iu, b_l0_iu, W_r0_iu,
           W_l1_ui, b_l1_ui, W_r1_ui, W_l1_iu, b_l1_iu, W_r1_iu,
           W_head1, b_head1, W_head2, b_head2, edge_index_ui, edge_index_iu):
    pad_n = E_PAD - E
    shape3 = (NS, CHUNKS_PER_TILE, CHUNK)
    src_ui = jnp.concatenate(
        [edge_index_ui[0], jnp.zeros((pad_n,), jnp.int32)])
    dst_ui = jnp.concatenate(
        [edge_index_ui[1], jnp.full((pad_n,), N_NODE, jnp.int32)])
    src_iu = jnp.concatenate(
        [edge_index_iu[0], jnp.zeros((pad_n,), jnp.int32)])
    dst_iu = jnp.concatenate(
        [edge_index_iu[1], jnp.full((pad_n,), N_NODE, jnp.int32)])
    zeros_blk = jnp.zeros((256, PIECE), jnp.float32)

    rec_it = _sc_counts(dst_ui.reshape(shape3))
    rec_us = _sc_counts(dst_iu.reshape(shape3))
    bkt_ui = _sc_bucket(src_ui, dst_ui)
    bkt_iu = _sc_bucket(src_iu, dst_iu)

    hu0, hi0 = _tc_proj(x_user, W_proj_user, b_proj_user.reshape(1, H),
                        x_item, W_proj_item, b_proj_item.reshape(1, H))

    def agg(h, bkt):
        s_p = _sc_aggregate(h.reshape(N_NODE * PPR, PIECE), *bkt, zeros_blk)
        return s_p.reshape(N_NODE, H)

    s_it = agg(hu0, bkt_ui)
    s_us = agg(hi0, bkt_iu)

    hi1, hu1 = _tc_layer0(s_it, rec_it, hi0, W_l0_ui, b_l0_ui.reshape(1, H),
                          W_r0_ui, s_us, rec_us, hu0, W_l0_iu,
                          b_l0_iu.reshape(1, H), W_r0_iu)

    s_us2 = agg(hi1, bkt_iu)

    return _tc_final(s_us2, rec_us, hu1, W_l1_iu, b_l1_iu.reshape(1, H),
                     W_r1_iu, W_head1, b_head1.reshape(1, H),
                     W_head2, b_head2.reshape(1, OUT))


# speedup vs baseline: 1.8576x; 1.5033x over previous
import functools

import jax
import jax.numpy as jnp
from jax import lax
from jax.experimental import pallas as pl
from jax.experimental.pallas import tpu as pltpu
from jax.experimental.pallas import tpu_sc as plsc

N_NODE = 10000
H = 256
OUT = 64
E = 160000
NC, NS, L = 2, 16, 16
HALF = N_NODE // NC
TRASH = HALF
ACC_ROWS = HALF + 8
PIECE = 16
PPR = H // PIECE
ACC_P = ACC_ROWS * PPR
BLKP = 8 * PPR
ZERO_BLOCKS = ACC_P // BLKP
CHUNK = 64
FLAT = CHUNK * PPR
SUBT = FLAT // 128
E_PAD = 163840
CHUNKS_PER_TILE = E_PAD // NS // CHUNK
OUT_BLOCKS = HALF * PPR // BLKP
CNT_N = 10240
CNT_SLICE = CNT_N // NS
REP_ROWS = 10016
ROWS_BLK = 1000


def _cnt_body(dst_hbm, rec_out, cnt_sh, dst_all, cnt_v, red_v, brow):
    c = lax.axis_index("c")
    s = lax.axis_index("s")
    zero16 = jnp.zeros((L,), jnp.float32)
    one16 = jnp.ones((L,), jnp.float32)
    iota = lax.iota(jnp.int32, L)

    pltpu.sync_copy(dst_hbm.at[s], dst_all)

    def zcnt(i, _):
        cnt_v[pl.ds(i * L, L)] = zero16
        return 0
    lax.fori_loop(0, CNT_N // L, zcnt, 0)

    def chunk_body(j, _):
        for i in range(CHUNK // L):
            d = dst_all[j, pl.ds(i * L, L)]
            plsc.addupdate_scatter(cnt_v, [d], one16)
        return 0
    lax.fori_loop(0, CHUNKS_PER_TILE, chunk_body, 0)

    pltpu.sync_copy(cnt_v, cnt_sh.at[s])
    plsc.subcore_barrier()
    r0 = s * CNT_SLICE
    for i in range(CNT_SLICE // L):
        red_v[pl.ds(i * L, L)] = zero16

    def red_body(t, _):
        pltpu.sync_copy(cnt_sh.at[t, pl.ds(r0, CNT_SLICE)], cnt_v.at[pl.ds(0, CNT_SLICE)])
        for i in range(CNT_SLICE // L):
            red_v[pl.ds(i * L, L)] = (red_v[pl.ds(i * L, L)]
                                      + cnt_v[pl.ds(i * L, L)])
        return 0
    lax.fori_loop(0, NS, red_body, 0)

    def rep_body(g, _):
        row0 = r0 + c * 320 + g * 8
        @pl.when(row0 < REP_ROWS)
        def _():
            for r in range(8):
                p = row0 + r - r0
                grp = (p // L) * L
                sl = 1.0 / jnp.maximum(red_v[pl.ds(grp, L)], 1.0)
                sel = jnp.where(iota == (p - grp), 1.0, 0.0)
                rec = jnp.sum(sl * sel)
                rv = jnp.full((L,), rec, jnp.float32)
                for q in range(128 // L):
                    brow[r, pl.ds(q * L, L)] = rv
            pltpu.sync_copy(brow, rec_out.at[pl.ds(row0, 8)])
        return 0
    lax.fori_loop(0, 320 // 8, rep_body, 0)


_sc_counts = functools.partial(
    pl.kernel,
    out_type=jax.ShapeDtypeStruct((REP_ROWS, 128), jnp.float32),
    mesh=plsc.VectorSubcoreMesh(core_axis_name="c", subcore_axis_name="s"),
    compiler_params=pltpu.CompilerParams(needs_layout_passes=False,
                                         use_tc_tiling_on_sc=False),
    scratch_types=[
        pltpu.VMEM_SHARED((NS, CNT_N), jnp.float32),
        pltpu.VMEM((CHUNKS_PER_TILE, CHUNK), jnp.int32),
        pltpu.VMEM((CNT_N,), jnp.float32),
        pltpu.VMEM((CNT_SLICE,), jnp.float32),
        pltpu.VMEM((8, 128), jnp.float32),
    ],
)(_cnt_body)


SLICE_E = E_PAD // (NC * NS)
CAP = SLICE_E + CHUNK


def _bucket_body(src_hbm, dst_hbm, bsrc_out, bdst_out, lens_out,
                 sv_all, dv_all, lsrc, ldst, hsrc, hdst, lenv):
    c = lax.axis_index("c")
    s = lax.axis_index("s")
    w = 2 * s + c
    zero16i = jnp.zeros((L,), jnp.int32)
    pad16 = jnp.full((L,), N_NODE, jnp.int32)

    pltpu.sync_copy(src_hbm.at[pl.ds(w * SLICE_E, SLICE_E)], sv_all)
    pltpu.sync_copy(dst_hbm.at[pl.ds(w * SLICE_E, SLICE_E)], dv_all)

    def grp(i, carry):
        lo, hi = carry
        sv = sv_all[pl.ds(i * L, L)]
        dv = dv_all[pl.ds(i * L, L)]
        mlow = dv < HALF
        plsc.store_compressed(lsrc.at[pl.ds(lo, L)], sv, mask=mlow)
        plsc.store_compressed(ldst.at[pl.ds(lo, L)], dv, mask=mlow)
        n_lo = plsc.all_reduce_population_count(mlow)
        mhigh = jnp.logical_not(mlow)
        plsc.store_compressed(hsrc.at[pl.ds(hi, L)], sv, mask=mhigh)
        plsc.store_compressed(hdst.at[pl.ds(hi, L)], dv, mask=mhigh)
        return (lo + jnp.max(n_lo), hi + (L - jnp.max(n_lo)))
    lo, hi = lax.fori_loop(0, SLICE_E // L, grp, (0, 0))

    def pad_tail(off, srcl, dstl):
        tgt = ((off + CHUNK - 1) // CHUNK) * CHUNK
        for i in range(CHUNK // L):
            @pl.when(off + i * L < tgt)
            def _():
                srcl[pl.ds(off + i * L, L)] = zero16i
                dstl[pl.ds(off + i * L, L)] = pad16
        return tgt // CHUNK
    nch_lo = pad_tail(lo, lsrc, ldst)
    nch_hi = pad_tail(hi, hsrc, hdst)

    pltpu.sync_copy(lsrc, bsrc_out.at[0, w])
    pltpu.sync_copy(ldst, bdst_out.at[0, w])
    pltpu.sync_copy(hsrc, bsrc_out.at[1, w])
    pltpu.sync_copy(hdst, bdst_out.at[1, w])
    lenv[pl.ds(0, L)] = jnp.full((L,), nch_lo, jnp.int32)
    lenv[pl.ds(L, L)] = jnp.full((L,), nch_hi, jnp.int32)
    pltpu.sync_copy(lenv.at[pl.ds(0, L)], lens_out.at[0, w])
    pltpu.sync_copy(lenv.at[pl.ds(L, L)], lens_out.at[1, w])


_sc_bucket = functools.partial(
    pl.kernel,
    out_type=(jax.ShapeDtypeStruct((NC, NC * NS, CAP), jnp.int32),
              jax.ShapeDtypeStruct((NC, NC * NS, CAP), jnp.int32),
              jax.ShapeDtypeStruct((NC, NC * NS, L), jnp.int32)),
    mesh=plsc.VectorSubcoreMesh(core_axis_name="c", subcore_axis_name="s"),
    compiler_params=pltpu.CompilerParams(needs_layout_passes=False,
                                         use_tc_tiling_on_sc=False),
    scratch_types=[
        pltpu.VMEM((SLICE_E,), jnp.int32),
        pltpu.VMEM((SLICE_E,), jnp.int32),
        pltpu.VMEM((CAP,), jnp.int32),
        pltpu.VMEM((CAP,), jnp.int32),
        pltpu.VMEM((CAP,), jnp.int32),
        pltpu.VMEM((CAP,), jnp.int32),
        pltpu.VMEM((2 * L,), jnp.int32),
    ],
)(_bucket_body)


def _agg_body(h_hbm, bsrc, bdst, lens, zeros_hbm, s_out,
              acc, src_v, dst_v, spidx, dpidx, rows, zb, lenv, gsem, ssem):
    c = lax.axis_index("c")
    s = lax.axis_index("s")
    base = c * HALF
    iota = lax.iota(jnp.int32, L)

    pltpu.sync_copy(zeros_hbm, zb)

    def zero_body(j, _):
        blk = j * NS + s
        @pl.when(blk < ZERO_BLOCKS)
        def _():
            pltpu.sync_copy(zb, acc.at[pl.ds(blk * BLKP, BLKP)])
        return 0
    lax.fori_loop(0, (ZERO_BLOCKS + NS - 1) // NS, zero_body, 0)
    plsc.subcore_barrier()

    lpg = 128 // PPR
    row16 = [(L * PPR // 128) * i + iota // lpg for i in range(CHUNK // L)]
    colbase = (iota % lpg) * PPR

    def sublist(w):
        pltpu.sync_copy(lens.at[c, w], lenv)
        nch = jnp.max(lenv[pl.ds(0, L)])

        def chunk_body(j, _):
            pltpu.sync_copy(bsrc.at[c, w, pl.ds(j * CHUNK, CHUNK)], src_v)
            pltpu.sync_copy(bdst.at[c, w, pl.ds(j * CHUNK, CHUNK)], dst_v)
            for i in range(CHUNK // L):
                sv = src_v[pl.ds(i * L, L)] * PPR
                d = dst_v[pl.ds(i * L, L)]
                dl = d - base
                ok = (dl >= 0) & (dl < HALF)
                dl = jnp.where(ok, dl, TRASH) * PPR
                for k in range(PPR):
                    col = colbase + k
                    plsc.store_scatter(spidx, [row16[i], col], sv + k)
                    plsc.store_scatter(dpidx, [row16[i], col], dl + k)
            gd = [pltpu.async_copy(h_hbm.at[spidx.at[t]],
                                   rows.at[pl.ds(t * 128, 128)], gsem)
                  for t in range(SUBT)]
            for dsc in gd:
                dsc.wait()
            sd = [pltpu.async_copy(rows.at[pl.ds(t * 128, 128)],
                                   acc.at[dpidx.at[t]], ssem, add=True)
                  for t in range(SUBT)]
            for dsc in sd:
                dsc.wait()
            return 0
        lax.fori_loop(0, nch, chunk_body, 0)

    sublist(2 * s)
    sublist(2 * s + 1)
    plsc.subcore_barrier()

    def out_body(j, _):
        blk = j * NS + s
        @pl.when(blk < OUT_BLOCKS)
        def _():
            pltpu.sync_copy(acc.at[pl.ds(blk * BLKP, BLKP)],
                            s_out.at[pl.ds(c * HALF * PPR + blk * BLKP,
                                           BLKP)])
        return 0
    lax.fori_loop(0, (OUT_BLOCKS + NS - 1) // NS, out_body, 0)


_sc_aggregate = functools.partial(
    pl.kernel,
    out_type=jax.ShapeDtypeStruct((N_NODE * PPR, PIECE), jnp.float32),
    mesh=plsc.VectorSubcoreMesh(core_axis_name="c", subcore_axis_name="s"),
    compiler_params=pltpu.CompilerParams(needs_layout_passes=False,
                                         use_tc_tiling_on_sc=False),
    scratch_types=[
        pltpu.VMEM_SHARED((ACC_P, PIECE), jnp.float32),
        pltpu.VMEM((CHUNK,), jnp.int32),
        pltpu.VMEM((CHUNK,), jnp.int32),
        pltpu.VMEM((SUBT, 128), jnp.int32),
        pltpu.VMEM((SUBT, 128), jnp.int32),
        pltpu.VMEM((FLAT, PIECE), jnp.float32),
        pltpu.VMEM((BLKP, PIECE), jnp.float32),
        pltpu.VMEM((L,), jnp.int32),
        pltpu.SemaphoreType.DMA,
        pltpu.SemaphoreType.DMA,
    ],
)(_agg_body)


def _proj_body(xu, wu, bu, xi, wi, bi, hu, hi):
    hu[:] = jnp.dot(xu[:], wu[:], preferred_element_type=jnp.float32) + bu[:]
    hi[:] = jnp.dot(xi[:], wi[:], preferred_element_type=jnp.float32) + bi[:]


def _layer0_body(sit, rit, hi0, wl_i, bl_i, wr_i,
                 sus, rus, hu0, wl_u, bl_u, wr_u, hi1, hu1):
    mi = sit[:] * rit[:, 0:1]
    hi1[:] = jax.nn.relu(
        jnp.dot(mi, wl_i[:], preferred_element_type=jnp.float32) + bl_i[:]
        + jnp.dot(hi0[:], wr_i[:], preferred_element_type=jnp.float32))
    mu = sus[:] * rus[:, 0:1]
    hu1[:] = jax.nn.relu(
        jnp.dot(mu, wl_u[:], preferred_element_type=jnp.float32) + bl_u[:]
        + jnp.dot(hu0[:], wr_u[:], preferred_element_type=jnp.float32))


def _final_body(sus2, rus, hu1, wl, bl, wr, wh1, bh1, wh2, bh2, out):
    mu = sus2[:] * rus[:, 0:1]
    hu2 = (jnp.dot(mu, wl[:], preferred_element_type=jnp.float32) + bl[:]
           + jnp.dot(hu1[:], wr[:], preferred_element_type=jnp.float32))
    t = jax.nn.relu(
        jnp.dot(hu2, wh1[:], preferred_element_type=jnp.float32) + bh1[:])
    out[:] = jnp.dot(t, wh2[:], preferred_element_type=jnp.float32) + bh2[:]


def _row_spec(cols):
    return pl.BlockSpec((ROWS_BLK, cols), lambda i: (i, 0))


def _full_spec(r, cols):
    return pl.BlockSpec((r, cols), lambda i: (0, 0))


_GRID = N_NODE // ROWS_BLK


def _tc_proj(xu, wu, bu, xi, wi, bi):
    return pl.pallas_call(
        _proj_body,
        grid=(_GRID,),
        in_specs=[_row_spec(256), _full_spec(256, H), _full_spec(1, H),
                  _row_spec(128), _full_spec(128, H), _full_spec(1, H)],
        out_specs=(_row_spec(H), _row_spec(H)),
        out_shape=(jax.ShapeDtypeStruct((N_NODE, H), jnp.float32),
                   jax.ShapeDtypeStruct((N_NODE, H), jnp.float32)),
    )(xu, wu, bu, xi, wi, bi)


def _tc_layer0(sit, rit, hi0, wl_i, bl_i, wr_i,
               sus, rus, hu0, wl_u, bl_u, wr_u):
    w = _full_spec(H, H)
    b = _full_spec(1, H)
    return pl.pallas_call(
        _layer0_body,
        grid=(_GRID,),
        in_specs=[_row_spec(H), _row_spec(128), _row_spec(H), w, b, w,
                  _row_spec(H), _row_spec(128), _row_spec(H), w, b, w],
        out_specs=(_row_spec(H), _row_spec(H)),
        out_shape=(jax.ShapeDtypeStruct((N_NODE, H), jnp.float32),
                   jax.ShapeDtypeStruct((N_NODE, H), jnp.float32)),
    )(sit, rit, hi0, wl_i, bl_i, wr_i, sus, rus, hu0, wl_u, bl_u, wr_u)


def _tc_final(sus2, rus, hu1, wl, bl, wr, wh1, bh1, wh2, bh2):
    w = _full_spec(H, H)
    b = _full_spec(1, H)
    return pl.pallas_call(
        _final_body,
        grid=(_GRID,),
        in_specs=[_row_spec(H), _row_spec(128), _row_spec(H), w, b, w,
                  w, b, _full_spec(H, OUT), _full_spec(1, OUT)],
        out_specs=_row_spec(OUT),
        out_shape=jax.ShapeDtypeStruct((N_NODE, OUT), jnp.float32),
    )(sus2, rus, hu1, wl, bl, wr, wh1, bh1, wh2, bh2)


def kernel(x_user, x_item, W_proj_user, b_proj_user, W_proj_item, b_proj_item,
           W_l0_ui, b_l0_ui, W_r0_ui, W_l0_iu, b_l0_iu, W_r0_iu,
           W_l1_ui, b_l1_ui, W_r1_ui, W_l1_iu, b_l1_iu, W_r1_iu,
           W_head1, b_head1, W_head2, b_head2, edge_index_ui, edge_index_iu):
    pad_n = E_PAD - E
    shape3 = (NS, CHUNKS_PER_TILE, CHUNK)
    src_ui = jnp.concatenate(
        [edge_index_ui[0], jnp.zeros((pad_n,), jnp.int32)])
    dst_ui = jnp.concatenate(
        [edge_index_ui[1], jnp.full((pad_n,), N_NODE, jnp.int32)])
    src_iu = jnp.concatenate(
        [edge_index_iu[0], jnp.zeros((pad_n,), jnp.int32)])
    dst_iu = jnp.concatenate(
        [edge_index_iu[1], jnp.full((pad_n,), N_NODE, jnp.int32)])
    zeros_blk = jnp.zeros((BLKP, PIECE), jnp.float32)

    rec_it = _sc_counts(dst_ui.reshape(shape3))
    rec_us = _sc_counts(dst_iu.reshape(shape3))
    bkt_ui = _sc_bucket(src_ui, dst_ui)
    bkt_iu = _sc_bucket(src_iu, dst_iu)

    hu0, hi0 = _tc_proj(x_user, W_proj_user, b_proj_user.reshape(1, H),
                        x_item, W_proj_item, b_proj_item.reshape(1, H))

    def agg(h, bkt):
        s_p = _sc_aggregate(h.reshape(N_NODE * PPR, PIECE), *bkt, zeros_blk)
        return s_p.reshape(N_NODE, H)

    s_it = agg(hu0, bkt_ui)
    s_us = agg(hi0, bkt_iu)

    hi1, hu1 = _tc_layer0(s_it, rec_it, hi0, W_l0_ui, b_l0_ui.reshape(1, H),
                          W_r0_ui, s_us, rec_us, hu0, W_l0_iu,
                          b_l0_iu.reshape(1, H), W_r0_iu)

    s_us2 = agg(hi1, bkt_iu)

    return _tc_final(s_us2, rec_us, hu1, W_l1_iu, b_l1_iu.reshape(1, H),
                     W_r1_iu, W_head1, b_head1.reshape(1, H),
                     W_head2, b_head2.reshape(1, OUT))


# speedup vs baseline: 2.0564x; 1.1071x over previous
import functools

import jax
import jax.numpy as jnp
from jax import lax
from jax.experimental import pallas as pl
from jax.experimental.pallas import tpu as pltpu
from jax.experimental.pallas import tpu_sc as plsc

N_NODE = 10000
H = 256
OUT = 64
E = 160000
NC, NS, L = 2, 16, 16
HALF = N_NODE // NC
TRASH = HALF
ACC_ROWS = HALF + 8
PIECE = 32
PPR = H // PIECE
ACC_P = ACC_ROWS * PPR
BLKP = 8 * PPR
ZERO_BLOCKS = ACC_P // BLKP
CHUNK = 64
FLAT = CHUNK * PPR
SUBT = FLAT // 128
E_PAD = 163840
CHUNKS_PER_TILE = E_PAD // NS // CHUNK
OUT_BLOCKS = HALF * PPR // BLKP
CNT_N = 10240
CNT_SLICE = CNT_N // NS
REP_ROWS = 10016
ROWS_BLK = 1000


def _cnt_body(dst_hbm, rec_out, cnt_sh, dst_all, cnt_v, red_v, brow):
    c = lax.axis_index("c")
    s = lax.axis_index("s")
    zero16 = jnp.zeros((L,), jnp.float32)
    one16 = jnp.ones((L,), jnp.float32)
    iota = lax.iota(jnp.int32, L)

    pltpu.sync_copy(dst_hbm.at[s], dst_all)

    def zcnt(i, _):
        cnt_v[pl.ds(i * L, L)] = zero16
        return 0
    lax.fori_loop(0, CNT_N // L, zcnt, 0)

    def chunk_body(j, _):
        for i in range(CHUNK // L):
            d = dst_all[j, pl.ds(i * L, L)]
            plsc.addupdate_scatter(cnt_v, [d], one16)
        return 0
    lax.fori_loop(0, CHUNKS_PER_TILE, chunk_body, 0)

    pltpu.sync_copy(cnt_v, cnt_sh.at[s])
    plsc.subcore_barrier()
    r0 = s * CNT_SLICE
    for i in range(CNT_SLICE // L):
        red_v[pl.ds(i * L, L)] = zero16

    def red_body(t, _):
        pltpu.sync_copy(cnt_sh.at[t, pl.ds(r0, CNT_SLICE)], cnt_v.at[pl.ds(0, CNT_SLICE)])
        for i in range(CNT_SLICE // L):
            red_v[pl.ds(i * L, L)] = (red_v[pl.ds(i * L, L)]
                                      + cnt_v[pl.ds(i * L, L)])
        return 0
    lax.fori_loop(0, NS, red_body, 0)

    def rep_body(g, _):
        row0 = r0 + c * 320 + g * 8
        @pl.when(row0 < REP_ROWS)
        def _():
            for r in range(8):
                p = row0 + r - r0
                grp = (p // L) * L
                sl = 1.0 / jnp.maximum(red_v[pl.ds(grp, L)], 1.0)
                sel = jnp.where(iota == (p - grp), 1.0, 0.0)
                rec = jnp.sum(sl * sel)
                rv = jnp.full((L,), rec, jnp.float32)
                for q in range(128 // L):
                    brow[r, pl.ds(q * L, L)] = rv
            pltpu.sync_copy(brow, rec_out.at[pl.ds(row0, 8)])
        return 0
    lax.fori_loop(0, 320 // 8, rep_body, 0)


_sc_counts = functools.partial(
    pl.kernel,
    out_type=jax.ShapeDtypeStruct((REP_ROWS, 128), jnp.float32),
    mesh=plsc.VectorSubcoreMesh(core_axis_name="c", subcore_axis_name="s"),
    compiler_params=pltpu.CompilerParams(needs_layout_passes=False,
                                         use_tc_tiling_on_sc=False),
    scratch_types=[
        pltpu.VMEM_SHARED((NS, CNT_N), jnp.float32),
        pltpu.VMEM((CHUNKS_PER_TILE, CHUNK), jnp.int32),
        pltpu.VMEM((CNT_N,), jnp.float32),
        pltpu.VMEM((CNT_SLICE,), jnp.float32),
        pltpu.VMEM((8, 128), jnp.float32),
    ],
)(_cnt_body)


SLICE_E = E_PAD // (NC * NS)
CAP = SLICE_E + CHUNK


def _bucket_body(src_hbm, dst_hbm, bsrc_out, bdst_out, lens_out,
                 sv_all, dv_all, lsrc, ldst, hsrc, hdst, lenv):
    c = lax.axis_index("c")
    s = lax.axis_index("s")
    w = 2 * s + c
    zero16i = jnp.zeros((L,), jnp.int32)
    pad16 = jnp.full((L,), N_NODE, jnp.int32)

    pltpu.sync_copy(src_hbm.at[pl.ds(w * SLICE_E, SLICE_E)], sv_all)
    pltpu.sync_copy(dst_hbm.at[pl.ds(w * SLICE_E, SLICE_E)], dv_all)

    def grp(i, carry):
        lo, hi = carry
        sv = sv_all[pl.ds(i * L, L)]
        dv = dv_all[pl.ds(i * L, L)]
        mlow = dv < HALF
        plsc.store_compressed(lsrc.at[pl.ds(lo, L)], sv, mask=mlow)
        plsc.store_compressed(ldst.at[pl.ds(lo, L)], dv, mask=mlow)
        n_lo = plsc.all_reduce_population_count(mlow)
        mhigh = jnp.logical_not(mlow)
        plsc.store_compressed(hsrc.at[pl.ds(hi, L)], sv, mask=mhigh)
        plsc.store_compressed(hdst.at[pl.ds(hi, L)], dv, mask=mhigh)
        return (lo + jnp.max(n_lo), hi + (L - jnp.max(n_lo)))
    lo, hi = lax.fori_loop(0, SLICE_E // L, grp, (0, 0))

    def pad_tail(off, srcl, dstl):
        tgt = ((off + CHUNK - 1) // CHUNK) * CHUNK
        for i in range(CHUNK // L):
            @pl.when(off + i * L < tgt)
            def _():
                srcl[pl.ds(off + i * L, L)] = zero16i
                dstl[pl.ds(off + i * L, L)] = pad16
        return tgt // CHUNK
    nch_lo = pad_tail(lo, lsrc, ldst)
    nch_hi = pad_tail(hi, hsrc, hdst)

    pltpu.sync_copy(lsrc, bsrc_out.at[0, w])
    pltpu.sync_copy(ldst, bdst_out.at[0, w])
    pltpu.sync_copy(hsrc, bsrc_out.at[1, w])
    pltpu.sync_copy(hdst, bdst_out.at[1, w])
    lenv[pl.ds(0, L)] = jnp.full((L,), nch_lo, jnp.int32)
    lenv[pl.ds(L, L)] = jnp.full((L,), nch_hi, jnp.int32)
    pltpu.sync_copy(lenv.at[pl.ds(0, L)], lens_out.at[0, w])
    pltpu.sync_copy(lenv.at[pl.ds(L, L)], lens_out.at[1, w])


_sc_bucket = functools.partial(
    pl.kernel,
    out_type=(jax.ShapeDtypeStruct((NC, NC * NS, CAP), jnp.int32),
              jax.ShapeDtypeStruct((NC, NC * NS, CAP), jnp.int32),
              jax.ShapeDtypeStruct((NC, NC * NS, L), jnp.int32)),
    mesh=plsc.VectorSubcoreMesh(core_axis_name="c", subcore_axis_name="s"),
    compiler_params=pltpu.CompilerParams(needs_layout_passes=False,
                                         use_tc_tiling_on_sc=False),
    scratch_types=[
        pltpu.VMEM((SLICE_E,), jnp.int32),
        pltpu.VMEM((SLICE_E,), jnp.int32),
        pltpu.VMEM((CAP,), jnp.int32),
        pltpu.VMEM((CAP,), jnp.int32),
        pltpu.VMEM((CAP,), jnp.int32),
        pltpu.VMEM((CAP,), jnp.int32),
        pltpu.VMEM((2 * L,), jnp.int32),
    ],
)(_bucket_body)


def _agg_body(h_hbm, bsrc, bdst, lens, zeros_hbm, s_out,
              acc, src_v, dst_v, spidx, dpidx, rows, zb, lenv, gsem, ssem):
    c = lax.axis_index("c")
    s = lax.axis_index("s")
    base = c * HALF
    iota = lax.iota(jnp.int32, L)

    pltpu.sync_copy(zeros_hbm, zb)

    def zero_body(j, _):
        blk = j * NS + s
        @pl.when(blk < ZERO_BLOCKS)
        def _():
            pltpu.sync_copy(zb, acc.at[pl.ds(blk * BLKP, BLKP)])
        return 0
    lax.fori_loop(0, (ZERO_BLOCKS + NS - 1) // NS, zero_body, 0)
    plsc.subcore_barrier()

    lpg = 128 // PPR
    row16 = [(L * PPR // 128) * i + iota // lpg for i in range(CHUNK // L)]
    colbase = (iota % lpg) * PPR

    def sublist(w):
        pltpu.sync_copy(lens.at[c, w], lenv)
        nch = jnp.max(lenv[pl.ds(0, L)])

        def chunk_body(j, _):
            pltpu.sync_copy(bsrc.at[c, w, pl.ds(j * CHUNK, CHUNK)], src_v)
            pltpu.sync_copy(bdst.at[c, w, pl.ds(j * CHUNK, CHUNK)], dst_v)
            for i in range(CHUNK // L):
                sv = src_v[pl.ds(i * L, L)] * PPR
                d = dst_v[pl.ds(i * L, L)]
                dl = d - base
                ok = (dl >= 0) & (dl < HALF)
                dl = jnp.where(ok, dl, TRASH) * PPR
                for k in range(PPR):
                    col = colbase + k
                    plsc.store_scatter(spidx, [row16[i], col], sv + k)
                    plsc.store_scatter(dpidx, [row16[i], col], dl + k)
            gd = [pltpu.async_copy(h_hbm.at[spidx.at[t]],
                                   rows.at[pl.ds(t * 128, 128)], gsem)
                  for t in range(SUBT)]
            for dsc in gd:
                dsc.wait()
            sd = [pltpu.async_copy(rows.at[pl.ds(t * 128, 128)],
                                   acc.at[dpidx.at[t]], ssem, add=True)
                  for t in range(SUBT)]
            for dsc in sd:
                dsc.wait()
            return 0
        lax.fori_loop(0, nch, chunk_body, 0)

    sublist(2 * s)
    sublist(2 * s + 1)
    plsc.subcore_barrier()

    def out_body(j, _):
        blk = j * NS + s
        @pl.when(blk < OUT_BLOCKS)
        def _():
            pltpu.sync_copy(acc.at[pl.ds(blk * BLKP, BLKP)],
                            s_out.at[pl.ds(c * HALF * PPR + blk * BLKP,
                                           BLKP)])
        return 0
    lax.fori_loop(0, (OUT_BLOCKS + NS - 1) // NS, out_body, 0)


_sc_aggregate = functools.partial(
    pl.kernel,
    out_type=jax.ShapeDtypeStruct((N_NODE * PPR, PIECE), jnp.float32),
    mesh=plsc.VectorSubcoreMesh(core_axis_name="c", subcore_axis_name="s"),
    compiler_params=pltpu.CompilerParams(needs_layout_passes=False,
                                         use_tc_tiling_on_sc=False),
    scratch_types=[
        pltpu.VMEM_SHARED((ACC_P, PIECE), jnp.float32),
        pltpu.VMEM((CHUNK,), jnp.int32),
        pltpu.VMEM((CHUNK,), jnp.int32),
        pltpu.VMEM((SUBT, 128), jnp.int32),
        pltpu.VMEM((SUBT, 128), jnp.int32),
        pltpu.VMEM((FLAT, PIECE), jnp.float32),
        pltpu.VMEM((BLKP, PIECE), jnp.float32),
        pltpu.VMEM((L,), jnp.int32),
        pltpu.SemaphoreType.DMA,
        pltpu.SemaphoreType.DMA,
    ],
)(_agg_body)


def _proj_body(xu, wu, bu, xi, wi, bi, hu, hi):
    hu[:] = jnp.dot(xu[:], wu[:], preferred_element_type=jnp.float32) + bu[:]
    hi[:] = jnp.dot(xi[:], wi[:], preferred_element_type=jnp.float32) + bi[:]


def _layer0_body(sit, rit, hi0, wl_i, bl_i, wr_i,
                 sus, rus, hu0, wl_u, bl_u, wr_u, hi1, hu1):
    mi = sit[:] * rit[:, 0:1]
    hi1[:] = jax.nn.relu(
        jnp.dot(mi, wl_i[:], preferred_element_type=jnp.float32) + bl_i[:]
        + jnp.dot(hi0[:], wr_i[:], preferred_element_type=jnp.float32))
    mu = sus[:] * rus[:, 0:1]
    hu1[:] = jax.nn.relu(
        jnp.dot(mu, wl_u[:], preferred_element_type=jnp.float32) + bl_u[:]
        + jnp.dot(hu0[:], wr_u[:], preferred_element_type=jnp.float32))


def _final_body(sus2, rus, hu1, wl, bl, wr, wh1, bh1, wh2, bh2, out):
    mu = sus2[:] * rus[:, 0:1]
    hu2 = (jnp.dot(mu, wl[:], preferred_element_type=jnp.float32) + bl[:]
           + jnp.dot(hu1[:], wr[:], preferred_element_type=jnp.float32))
    t = jax.nn.relu(
        jnp.dot(hu2, wh1[:], preferred_element_type=jnp.float32) + bh1[:])
    out[:] = jnp.dot(t, wh2[:], preferred_element_type=jnp.float32) + bh2[:]


def _row_spec(cols):
    return pl.BlockSpec((ROWS_BLK, cols), lambda i: (i, 0))


def _full_spec(r, cols):
    return pl.BlockSpec((r, cols), lambda i: (0, 0))


_GRID = N_NODE // ROWS_BLK


def _tc_proj(xu, wu, bu, xi, wi, bi):
    return pl.pallas_call(
        _proj_body,
        grid=(_GRID,),
        in_specs=[_row_spec(256), _full_spec(256, H), _full_spec(1, H),
                  _row_spec(128), _full_spec(128, H), _full_spec(1, H)],
        out_specs=(_row_spec(H), _row_spec(H)),
        out_shape=(jax.ShapeDtypeStruct((N_NODE, H), jnp.float32),
                   jax.ShapeDtypeStruct((N_NODE, H), jnp.float32)),
    )(xu, wu, bu, xi, wi, bi)


def _tc_layer0(sit, rit, hi0, wl_i, bl_i, wr_i,
               sus, rus, hu0, wl_u, bl_u, wr_u):
    w = _full_spec(H, H)
    b = _full_spec(1, H)
    return pl.pallas_call(
        _layer0_body,
        grid=(_GRID,),
        in_specs=[_row_spec(H), _row_spec(128), _row_spec(H), w, b, w,
                  _row_spec(H), _row_spec(128), _row_spec(H), w, b, w],
        out_specs=(_row_spec(H), _row_spec(H)),
        out_shape=(jax.ShapeDtypeStruct((N_NODE, H), jnp.float32),
                   jax.ShapeDtypeStruct((N_NODE, H), jnp.float32)),
    )(sit, rit, hi0, wl_i, bl_i, wr_i, sus, rus, hu0, wl_u, bl_u, wr_u)


def _tc_final(sus2, rus, hu1, wl, bl, wr, wh1, bh1, wh2, bh2):
    w = _full_spec(H, H)
    b = _full_spec(1, H)
    return pl.pallas_call(
        _final_body,
        grid=(_GRID,),
        in_specs=[_row_spec(H), _row_spec(128), _row_spec(H), w, b, w,
                  w, b, _full_spec(H, OUT), _full_spec(1, OUT)],
        out_specs=_row_spec(OUT),
        out_shape=jax.ShapeDtypeStruct((N_NODE, OUT), jnp.float32),
    )(sus2, rus, hu1, wl, bl, wr, wh1, bh1, wh2, bh2)


def kernel(x_user, x_item, W_proj_user, b_proj_user, W_proj_item, b_proj_item,
           W_l0_ui, b_l0_ui, W_r0_ui, W_l0_iu, b_l0_iu, W_r0_iu,
           W_l1_ui, b_l1_ui, W_r1_ui, W_l1_iu, b_l1_iu, W_r1_iu,
           W_head1, b_head1, W_head2, b_head2, edge_index_ui, edge_index_iu):
    pad_n = E_PAD - E
    shape3 = (NS, CHUNKS_PER_TILE, CHUNK)
    src_ui = jnp.concatenate(
        [edge_index_ui[0], jnp.zeros((pad_n,), jnp.int32)])
    dst_ui = jnp.concatenate(
        [edge_index_ui[1], jnp.full((pad_n,), N_NODE, jnp.int32)])
    src_iu = jnp.concatenate(
        [edge_index_iu[0], jnp.zeros((pad_n,), jnp.int32)])
    dst_iu = jnp.concatenate(
        [edge_index_iu[1], jnp.full((pad_n,), N_NODE, jnp.int32)])
    zeros_blk = jnp.zeros((BLKP, PIECE), jnp.float32)

    rec_it = _sc_counts(dst_ui.reshape(shape3))
    rec_us = _sc_counts(dst_iu.reshape(shape3))
    bkt_ui = _sc_bucket(src_ui, dst_ui)
    bkt_iu = _sc_bucket(src_iu, dst_iu)

    hu0, hi0 = _tc_proj(x_user, W_proj_user, b_proj_user.reshape(1, H),
                        x_item, W_proj_item, b_proj_item.reshape(1, H))

    def agg(h, bkt):
        s_p = _sc_aggregate(h.reshape(N_NODE * PPR, PIECE), *bkt, zeros_blk)
        return s_p.reshape(N_NODE, H)

    s_it = agg(hu0, bkt_ui)
    s_us = agg(hi0, bkt_iu)

    hi1, hu1 = _tc_layer0(s_it, rec_it, hi0, W_l0_ui, b_l0_ui.reshape(1, H),
                          W_r0_ui, s_us, rec_us, hu0, W_l0_iu,
                          b_l0_iu.reshape(1, H), W_r0_iu)

    s_us2 = agg(hi1, bkt_iu)

    return _tc_final(s_us2, rec_us, hu1, W_l1_iu, b_l1_iu.reshape(1, H),
                     W_r1_iu, W_head1, b_head1.reshape(1, H),
                     W_head2, b_head2.reshape(1, OUT))
